# baseline (speedup 1.0000x reference)
import functools

import jax
import jax.numpy as jnp
from jax import lax
from jax.experimental import pallas as pl
from jax.experimental.pallas import tpu as pltpu
from jax.experimental.pallas import tpu_sc as plsc

_B, _S, _D = 4, 8192, 1024
_NW = 32
_ROWS_PER_W = (_B * _S) // _NW
_W_PER_BATCH = _S // _ROWS_PER_W
_CHUNK_ROWS = 16
_CHUNK_W = _CHUNK_ROWS * _D
_N_CHUNKS = _ROWS_PER_W // _CHUNK_ROWS

_mesh = plsc.VectorSubcoreMesh(core_axis_name="c", subcore_axis_name="s")


@functools.partial(
    pl.kernel,
    mesh=_mesh,
    out_type=jax.ShapeDtypeStruct((_B * _S * _D,), jnp.float32),
    scratch_types=[
        pltpu.VMEM((_CHUNK_W,), jnp.float32),
        pltpu.VMEM((_CHUNK_W,), jnp.float32),
    ],
)
def _sc_add(x_hbm, pos_hbm, out_hbm, x_v, p_v):
    wid = lax.axis_index("s") * 2 + lax.axis_index("c")
    base = wid * (_ROWS_PER_W * _D)
    pbase = lax.rem(wid, _W_PER_BATCH) * (_ROWS_PER_W * _D)

    def body(c, carry):
        off = pl.multiple_of(base + c * _CHUNK_W, 8)
        poff = pl.multiple_of(pbase + c * _CHUNK_W, 8)
        pltpu.sync_copy(x_hbm.at[pl.ds(off, _CHUNK_W)], x_v)
        pltpu.sync_copy(pos_hbm.at[pl.ds(poff, _CHUNK_W)], p_v)

        def inner(i, c2):
            s = pl.ds(i * 16, 16)
            x_v[s] = x_v[s] + p_v[s]
            return c2

        lax.fori_loop(0, _CHUNK_W // 16, inner, 0)
        pltpu.sync_copy(x_v, out_hbm.at[pl.ds(off, _CHUNK_W)])
        return carry

    lax.fori_loop(0, _N_CHUNKS, body, 0)


def kernel(x, pos_table):
    out = _sc_add(x.reshape(-1), pos_table.reshape(-1))
    return out.reshape(_B, _S, _D)


# speedup vs baseline: 1.0182x; 1.0182x over previous
import functools

import jax
import jax.numpy as jnp
from jax import lax
from jax.experimental import pallas as pl
from jax.experimental.pallas import tpu as pltpu
from jax.experimental.pallas import tpu_sc as plsc

_B, _S, _D = 4, 8192, 1024
_NW = 32
_ROWS_PER_W = (_B * _S) // _NW
_W_PER_BATCH = _S // _ROWS_PER_W
_CHUNK_ROWS = 16
_CHUNK_W = _CHUNK_ROWS * _D
_N_CHUNKS = _ROWS_PER_W // _CHUNK_ROWS
_UNROLL = 8

_mesh = plsc.VectorSubcoreMesh(core_axis_name="c", subcore_axis_name="s")


@functools.partial(
    pl.kernel,
    mesh=_mesh,
    out_type=jax.ShapeDtypeStruct((_B * _S * _D,), jnp.float32),
    scratch_types=[
        pltpu.VMEM((2, _CHUNK_W), jnp.float32),
        pltpu.VMEM((2, _CHUNK_W), jnp.float32),
        pltpu.SemaphoreType.DMA((2,)),
        pltpu.SemaphoreType.DMA((2,)),
        pltpu.SemaphoreType.DMA((2,)),
    ],
)
def _sc_add(x_hbm, pos_hbm, out_hbm, x_v, p_v, xsem, psem, osem):
    wid = lax.axis_index("s") * 2 + lax.axis_index("c")
    base = wid * (_ROWS_PER_W * _D)
    pbase = lax.rem(wid, _W_PER_BATCH) * (_ROWS_PER_W * _D)

    def off_of(c):
        return pl.multiple_of(base + c * _CHUNK_W, 8)

    def poff_of(c):
        return pl.multiple_of(pbase + c * _CHUNK_W, 8)

    def start_in(c, k):
        pltpu.make_async_copy(x_hbm.at[pl.ds(off_of(c), _CHUNK_W)],
                              x_v.at[k], xsem.at[k]).start()
        pltpu.make_async_copy(pos_hbm.at[pl.ds(poff_of(c), _CHUNK_W)],
                              p_v.at[k], psem.at[k]).start()

    def wait_in(c, k):
        pltpu.make_async_copy(x_hbm.at[pl.ds(off_of(c), _CHUNK_W)],
                              x_v.at[k], xsem.at[k]).wait()
        pltpu.make_async_copy(pos_hbm.at[pl.ds(poff_of(c), _CHUNK_W)],
                              p_v.at[k], psem.at[k]).wait()

    def start_out(c, k):
        pltpu.make_async_copy(x_v.at[k], out_hbm.at[pl.ds(off_of(c), _CHUNK_W)],
                              osem.at[k]).start()

    def wait_out(c, k):
        pltpu.make_async_copy(x_v.at[k], out_hbm.at[pl.ds(off_of(c), _CHUNK_W)],
                              osem.at[k]).wait()

    start_in(0, 0)

    def body(g, carry):
        k = lax.rem(g, 2)
        kn = lax.rem(g + 1, 2)
        nxt = jnp.minimum(g + 1, _N_CHUNKS - 1)

        @pl.when(g >= 1)
        def _():
            wait_out(g - 1, kn)

        @pl.when(g + 1 < _N_CHUNKS)
        def _():
            start_in(nxt, kn)

        wait_in(g, k)

        def inner(i, c2):
            for j in range(_UNROLL):
                s = pl.ds((i * _UNROLL + j) * 16, 16)
                x_v[k, s] = x_v[k, s] + p_v[k, s]
            return c2

        lax.fori_loop(0, _CHUNK_W // (16 * _UNROLL), inner, 0)
        start_out(g, k)
        return carry

    lax.fori_loop(0, _N_CHUNKS, body, 0)
    wait_out(_N_CHUNKS - 1, (_N_CHUNKS - 1) % 2)


def kernel(x, pos_table):
    out = _sc_add(x.reshape(-1), pos_table.reshape(-1))
    return out.reshape(_B, _S, _D)


# speedup vs baseline: 2.2180x; 2.1783x over previous
import functools

import jax
import jax.numpy as jnp
from jax import lax
from jax.experimental import pallas as pl
from jax.experimental.pallas import tpu as pltpu
from jax.experimental.pallas import tpu_sc as plsc

_B, _S, _D = 4, 8192, 1024
_NW = 32
_ROWS_PER_W = (_B * _S) // _NW
_W_PER_BATCH = _S // _ROWS_PER_W
_CHUNK_ROWS = 16
_N_CHUNKS = _ROWS_PER_W // _CHUNK_ROWS
_NBUF = 3

_mesh = plsc.VectorSubcoreMesh(core_axis_name="c", subcore_axis_name="s")


@functools.partial(
    pl.kernel,
    mesh=_mesh,
    out_type=jax.ShapeDtypeStruct((_B, _S, _D), jnp.float32),
    scratch_types=[
        pltpu.VMEM((_NBUF, _CHUNK_ROWS, _D), jnp.float32),
        pltpu.VMEM((_NBUF, _CHUNK_ROWS, _D), jnp.float32),
        pltpu.SemaphoreType.DMA((_NBUF,)),
        pltpu.SemaphoreType.DMA((_NBUF,)),
        pltpu.SemaphoreType.DMA((_NBUF,)),
    ],
)
def _sc_add(x_hbm, pos_hbm, out_hbm, x_v, p_v, xsem, psem, osem):
    wid = lax.axis_index("s") * 2 + lax.axis_index("c")
    b = wid // _W_PER_BATCH
    row0 = lax.rem(wid, _W_PER_BATCH) * _ROWS_PER_W

    def x_slice(c):
        return x_hbm.at[b, pl.ds(row0 + c * _CHUNK_ROWS, _CHUNK_ROWS)]

    def p_slice(c):
        return pos_hbm.at[pl.ds(row0 + c * _CHUNK_ROWS, _CHUNK_ROWS)]

    def o_slice(c):
        return out_hbm.at[b, pl.ds(row0 + c * _CHUNK_ROWS, _CHUNK_ROWS)]

    def start_in(c, k):
        pltpu.make_async_copy(x_slice(c), x_v.at[k], xsem.at[k]).start()
        pltpu.make_async_copy(p_slice(c), p_v.at[k], psem.at[k]).start()

    def wait_in(c, k):
        pltpu.make_async_copy(x_slice(c), x_v.at[k], xsem.at[k]).wait()
        pltpu.make_async_copy(p_slice(c), p_v.at[k], psem.at[k]).wait()

    def start_out(c, k):
        pltpu.make_async_copy(x_v.at[k], o_slice(c), osem.at[k]).start()

    def wait_out(c, k):
        pltpu.make_async_copy(x_v.at[k], o_slice(c), osem.at[k]).wait()

    start_in(0, 0)
    start_in(1, 1)

    def body(g, carry):
        k = lax.rem(g, _NBUF)
        k2 = lax.rem(g + 2, _NBUF)
        wait_in(g, k)

        def row(r, c2):
            for j in range(_D // 16):
                s = pl.ds(j * 16, 16)
                x_v[k, r, s] = x_v[k, r, s] + p_v[k, r, s]
            return c2

        lax.fori_loop(0, _CHUNK_ROWS, row, 0)
        start_out(g, k)

        @pl.when(g < _N_CHUNKS - 2)
        def _():
            @pl.when(g >= 1)
            def _():
                wait_out(g - 1, k2)

            start_in(g + 2, k2)

        return carry

    lax.fori_loop(0, _N_CHUNKS, body, 0)
    for c in (_N_CHUNKS - 3, _N_CHUNKS - 2, _N_CHUNKS - 1):
        wait_out(c, c % _NBUF)


def kernel(x, pos_table):
    return _sc_add(x, pos_table)


# speedup vs baseline: 4.9696x; 2.2405x over previous
import functools

import jax
import jax.numpy as jnp
from jax import lax
from jax.experimental import pallas as pl
from jax.experimental.pallas import tpu as pltpu
from jax.experimental.pallas import tpu_sc as plsc

_B, _S, _D = 4, 8192, 1024
_NW = 32
_ROWS_PER_W = (_B * _S) // _NW
_W_PER_BATCH = _S // _ROWS_PER_W
_CHUNK_ROWS = 16
_N_CHUNKS = _ROWS_PER_W // _CHUNK_ROWS

_mesh = plsc.VectorSubcoreMesh(core_axis_name="c", subcore_axis_name="s")


@functools.partial(
    pl.kernel,
    mesh=_mesh,
    out_type=jax.ShapeDtypeStruct((_B, _S, _D), jnp.float32),
    scratch_types=[
        pltpu.VMEM((2, _CHUNK_ROWS, _D), jnp.float32),
        pltpu.VMEM((2, _CHUNK_ROWS, _D), jnp.float32),
        pltpu.VMEM((2, _CHUNK_ROWS, _D), jnp.float32),
        pltpu.SemaphoreType.DMA((2,)),
        pltpu.SemaphoreType.DMA((2,)),
        pltpu.SemaphoreType.DMA((2,)),
    ],
)
def _sc_add(x_hbm, pos_hbm, out_hbm, x_v, p_v, o_v, xsem, psem, osem):
    wid = lax.axis_index("s") * 2 + lax.axis_index("c")
    b = wid // _W_PER_BATCH
    row0 = lax.rem(wid, _W_PER_BATCH) * _ROWS_PER_W

    def x_slice(c):
        return x_hbm.at[b, pl.ds(row0 + c * _CHUNK_ROWS, _CHUNK_ROWS)]

    def p_slice(c):
        return pos_hbm.at[pl.ds(row0 + c * _CHUNK_ROWS, _CHUNK_ROWS)]

    def o_slice(c):
        return out_hbm.at[b, pl.ds(row0 + c * _CHUNK_ROWS, _CHUNK_ROWS)]

    def start_in(c, k):
        pltpu.make_async_copy(x_slice(c), x_v.at[k], xsem.at[k]).start()
        pltpu.make_async_copy(p_slice(c), p_v.at[k], psem.at[k]).start()

    def wait_in(c, k):
        pltpu.make_async_copy(x_slice(c), x_v.at[k], xsem.at[k]).wait()
        pltpu.make_async_copy(p_slice(c), p_v.at[k], psem.at[k]).wait()

    def start_out(c, k):
        pltpu.make_async_copy(o_v.at[k], o_slice(c), osem.at[k]).start()

    def wait_out(c, k):
        pltpu.make_async_copy(o_v.at[k], o_slice(c), osem.at[k]).wait()

    start_in(0, 0)

    def body(g, carry):
        k = lax.rem(g, 2)
        kn = lax.rem(g + 1, 2)

        @pl.when(g + 1 < _N_CHUNKS)
        def _():
            start_in(g + 1, kn)

        wait_in(g, k)

        @pl.when(g >= 2)
        def _():
            wait_out(g - 2, k)

        @plsc.parallel_loop(0, _CHUNK_ROWS * _D, 16, unroll=8)
        def _(i):
            r = lax.shift_right_logical(i, 10)
            s = pl.ds(pl.multiple_of(lax.bitwise_and(i, _D - 1), 16), 16)
            o_v[k, r, s] = x_v[k, r, s] + p_v[k, r, s]
        start_out(g, k)
        return carry

    lax.fori_loop(0, _N_CHUNKS, body, 0)
    wait_out(_N_CHUNKS - 2, (_N_CHUNKS - 2) % 2)
    wait_out(_N_CHUNKS - 1, (_N_CHUNKS - 1) % 2)


def kernel(x, pos_table):
    return _sc_add(x, pos_table)


# speedup vs baseline: 6.0880x; 1.2250x over previous
import functools

import jax
import jax.numpy as jnp
from jax import lax
from jax.experimental import pallas as pl
from jax.experimental.pallas import tpu as pltpu
from jax.experimental.pallas import tpu_sc as plsc

_B, _S, _D = 4, 8192, 1024
_NW = 32
_SEQ_PER_W = _S // _NW
_CHUNK_ROWS = 16
_NPC = _SEQ_PER_W // _CHUNK_ROWS
_NCH = _NPC * _B

_mesh = plsc.VectorSubcoreMesh(core_axis_name="c", subcore_axis_name="s")


@functools.partial(
    pl.kernel,
    mesh=_mesh,
    out_type=jax.ShapeDtypeStruct((_B, _S, _D), jnp.float32),
    scratch_types=[
        pltpu.VMEM((3, _CHUNK_ROWS, _D), jnp.float32),
        pltpu.VMEM((2, _CHUNK_ROWS, _D), jnp.float32),
        pltpu.SemaphoreType.DMA((3,)),
        pltpu.SemaphoreType.DMA((2,)),
        pltpu.SemaphoreType.DMA((3,)),
    ],
)
def _sc_add(x_hbm, pos_hbm, out_hbm, x_v, p_v, xsem, psem, osem):
    wid = lax.axis_index("s") * 2 + lax.axis_index("c")
    seq0 = wid * _SEQ_PER_W

    def x_slice(c):
        return x_hbm.at[lax.rem(c, _B),
                        pl.ds(seq0 + (c // _B) * _CHUNK_ROWS, _CHUNK_ROWS)]

    def o_slice(c):
        return out_hbm.at[lax.rem(c, _B),
                          pl.ds(seq0 + (c // _B) * _CHUNK_ROWS, _CHUNK_ROWS)]

    def p_slice(pc):
        return pos_hbm.at[pl.ds(seq0 + pc * _CHUNK_ROWS, _CHUNK_ROWS)]

    def start_in(c, k):
        pltpu.make_async_copy(x_slice(c), x_v.at[k], xsem.at[k]).start()

    def wait_in(c, k):
        pltpu.make_async_copy(x_slice(c), x_v.at[k], xsem.at[k]).wait()

    def start_p(pc, kp):
        pltpu.make_async_copy(p_slice(pc), p_v.at[kp], psem.at[kp]).start()

    def wait_p(pc, kp):
        pltpu.make_async_copy(p_slice(pc), p_v.at[kp], psem.at[kp]).wait()

    def start_out(c, k):
        pltpu.make_async_copy(x_v.at[k], o_slice(c), osem.at[k]).start()

    def wait_out(c, k):
        pltpu.make_async_copy(x_v.at[k], o_slice(c), osem.at[k]).wait()

    start_in(0, 0)
    start_in(1, 1)
    start_p(0, 0)
    start_p(1, 1)

    def body(g, carry):
        k = lax.rem(g, 3)
        b = lax.rem(g, _B)
        pc = g // _B
        kp = lax.rem(pc, 2)

        @pl.when(b == 0)
        def _():
            wait_p(pc, kp)

            @pl.when(jnp.logical_and(pc >= 1, pc + 1 < _NPC))
            def _():
                start_p(pc + 1, lax.rem(pc + 1, 2))

        wait_in(g, k)

        @plsc.parallel_loop(0, _CHUNK_ROWS * _D, 16, unroll=8)
        def _(i):
            r = lax.shift_right_logical(i, 10)
            s = pl.ds(pl.multiple_of(lax.bitwise_and(i, _D - 1), 16), 16)
            x_v[k, r, s] = x_v[k, r, s] + p_v[kp, r, s]

        start_out(g, k)

        @pl.when(g + 2 < _NCH)
        def _():
            @pl.when(g >= 1)
            def _():
                wait_out(g - 1, lax.rem(g + 2, 3))

            start_in(g + 2, lax.rem(g + 2, 3))

        return carry

    lax.fori_loop(0, _NCH, body, 0)
    for c in (_NCH - 3, _NCH - 2, _NCH - 1):
        wait_out(c, c % 3)


def kernel(x, pos_table):
    return _sc_add(x, pos_table)
